# baseline (speedup 1.0000x reference)
import jax
import jax.numpy as jnp
from jax.experimental import pallas as pl
from jax.experimental.pallas import tpu as pltpu

_V = 200
_BLOCK_B = 32
_CHUNKS = 4


def _fused_kernel(idx_ref, tgt_ref, emb_ref, logits_ref, losssum_ref):
    idx_blk = idx_ref[...].astype(jnp.float32)
    tgt_blk = tgt_ref[...].astype(jnp.float32)
    emb = emb_ref[...]
    rb, T = idx_blk.shape
    n = rb * T
    v_pad = emb.shape[1]

    row = jax.lax.broadcasted_iota(jnp.int32, (n, 1), 0)
    colb = jax.lax.broadcasted_iota(jnp.int32, (n, rb), 1)
    sel = ((row // T) == colb).astype(jnp.float32)
    colt = jax.lax.broadcasted_iota(jnp.int32, (n, T), 1)
    tmask = colt == (row % T)

    rows_idx = jnp.dot(sel, idx_blk, preferred_element_type=jnp.float32)
    idx_flat = jnp.sum(jnp.where(tmask, rows_idx, 0.0), axis=1, keepdims=True)
    rows_tgt = jnp.dot(sel, tgt_blk, preferred_element_type=jnp.float32)
    tgt_flat = jnp.sum(jnp.where(tmask, rows_tgt, 0.0), axis=1, keepdims=True)

    colv = jax.lax.broadcasted_iota(jnp.int32, (n, v_pad), 1)
    onehot = (colv == idx_flat.astype(jnp.int32)).astype(jnp.float32)
    logits = jnp.dot(onehot, emb, preferred_element_type=jnp.float32)
    logits_ref[...] = logits

    m = jnp.max(emb, axis=1, keepdims=True)
    lse_vec = m + jnp.log(jnp.sum(jnp.exp(emb - m), axis=1, keepdims=True))
    row_lse = jnp.dot(onehot, lse_vec, preferred_element_type=jnp.float32)

    tgt_logit = jnp.sum(jnp.where(colv == tgt_flat.astype(jnp.int32),
                                  logits, 0.0), axis=1, keepdims=True)
    losssum_ref[...] = jnp.sum(row_lse - tgt_logit, keepdims=True)[None]


@jax.jit
def kernel(idx, targets, emb_padded):
    B, T = idx.shape
    V_pad = emb_padded.shape[1]
    N = B * T
    tile_n = _BLOCK_B * T
    tiles_per_chunk = B // (_BLOCK_B * _CHUNKS)
    rows_per_chunk = tiles_per_chunk * tile_n

    cost = pl.CostEstimate(
        flops=2 * (N // _CHUNKS) * V_pad * V_pad,
        transcendentals=tiles_per_chunk * V_pad * V_pad,
        bytes_accessed=(2 * (N // _CHUNKS) * 4 + V_pad * V_pad * 4
                        + (N // _CHUNKS) * V_pad * 4))

    logit_chunks, loss_total = [], jnp.float32(0.0)
    for c in range(_CHUNKS):
        base = c * tiles_per_chunk
        logits_c, loss_sums_c = pl.pallas_call(
            _fused_kernel,
            out_shape=(
                jax.ShapeDtypeStruct((rows_per_chunk, V_pad), jnp.float32),
                jax.ShapeDtypeStruct((tiles_per_chunk, 1, 1), jnp.float32),
            ),
            grid=(tiles_per_chunk,),
            in_specs=[
                pl.BlockSpec((_BLOCK_B, T), lambda i, b=base: (b + i, 0)),
                pl.BlockSpec((_BLOCK_B, T), lambda i, b=base: (b + i, 0)),
                pl.BlockSpec((V_pad, V_pad), lambda i: (0, 0)),
            ],
            out_specs=(
                pl.BlockSpec((tile_n, V_pad), lambda i: (i, 0)),
                pl.BlockSpec((1, 1, 1), lambda i: (i, 0, 0)),
            ),
            compiler_params=pltpu.CompilerParams(
                dimension_semantics=("parallel",),
                vmem_limit_bytes=64 * 1024 * 1024,
            ),
            cost_estimate=cost,
        )(idx, targets, emb_padded)
        logit_chunks.append(logits_c[:, :_V])
        loss_total = loss_total + jnp.sum(loss_sums_c)

    logits = jnp.concatenate(logit_chunks, axis=0)
    loss = loss_total / jnp.float32(N)
    return logits, loss


# speedup vs baseline: 1.2116x; 1.2116x over previous
import jax
import jax.numpy as jnp
from jax.experimental import pallas as pl
from jax.experimental.pallas import tpu as pltpu

_V = 200
_BLOCK_B = 32


def _fused_kernel(idx_ref, tgt_ref, emb_ref, logits_ref, losssum_ref):
    idx_blk = idx_ref[...].astype(jnp.float32)
    tgt_blk = tgt_ref[...].astype(jnp.float32)
    emb = emb_ref[...]
    rb, T = idx_blk.shape
    n = rb * T
    v_pad = emb.shape[1]

    row = jax.lax.broadcasted_iota(jnp.int32, (n, 1), 0)
    colb = jax.lax.broadcasted_iota(jnp.int32, (n, rb), 1)
    sel = ((row // T) == colb).astype(jnp.float32)
    colt = jax.lax.broadcasted_iota(jnp.int32, (n, T), 1)
    tmask = colt == (row % T)

    rows_idx = jnp.dot(sel, idx_blk, preferred_element_type=jnp.float32)
    idx_flat = jnp.sum(jnp.where(tmask, rows_idx, 0.0), axis=1, keepdims=True)
    rows_tgt = jnp.dot(sel, tgt_blk, preferred_element_type=jnp.float32)
    tgt_flat = jnp.sum(jnp.where(tmask, rows_tgt, 0.0), axis=1, keepdims=True)

    colv = jax.lax.broadcasted_iota(jnp.int32, (n, v_pad), 1)
    onehot = (colv == idx_flat.astype(jnp.int32)).astype(jnp.float32)
    logits = jnp.dot(onehot, emb, preferred_element_type=jnp.float32)
    logits_ref[...] = logits.astype(jnp.bfloat16)

    m = jnp.max(emb, axis=1, keepdims=True)
    lse_vec = m + jnp.log(jnp.sum(jnp.exp(emb - m), axis=1, keepdims=True))
    row_lse = jnp.dot(onehot, lse_vec, preferred_element_type=jnp.float32)

    tgt_logit = jnp.sum(jnp.where(colv == tgt_flat.astype(jnp.int32),
                                  logits, 0.0), axis=1, keepdims=True)
    losssum_ref[...] = jnp.sum(row_lse - tgt_logit, keepdims=True)[None]


@jax.jit
def kernel(idx, targets, emb_padded):
    B, T = idx.shape
    V_pad = emb_padded.shape[1]
    N = B * T
    tile_n = _BLOCK_B * T
    num_tiles = B // _BLOCK_B

    cost = pl.CostEstimate(
        flops=2 * N * V_pad * V_pad,
        transcendentals=num_tiles * V_pad * V_pad,
        bytes_accessed=2 * N * 4 + V_pad * V_pad * 4 + N * V_pad * 2)
    logits, loss_sums = pl.pallas_call(
        _fused_kernel,
        out_shape=(
            jax.ShapeDtypeStruct((N, V_pad), jnp.bfloat16),
            jax.ShapeDtypeStruct((num_tiles, 1, 1), jnp.float32),
        ),
        grid=(num_tiles,),
        in_specs=[
            pl.BlockSpec((_BLOCK_B, T), lambda i: (i, 0)),
            pl.BlockSpec((_BLOCK_B, T), lambda i: (i, 0)),
            pl.BlockSpec((V_pad, V_pad), lambda i: (0, 0)),
        ],
        out_specs=(
            pl.BlockSpec((tile_n, V_pad), lambda i: (i, 0)),
            pl.BlockSpec((1, 1, 1), lambda i: (i, 0, 0)),
        ),
        compiler_params=pltpu.CompilerParams(
            dimension_semantics=("parallel",),
            vmem_limit_bytes=64 * 1024 * 1024,
        ),
        cost_estimate=cost,
    )(idx, targets, emb_padded)

    loss = jnp.sum(loss_sums) / jnp.float32(N)
    return logits[:, :_V].astype(jnp.float32), loss


# speedup vs baseline: 1.6270x; 1.3428x over previous
import jax
import jax.numpy as jnp
from jax.experimental import pallas as pl
from jax.experimental.pallas import tpu as pltpu

_V = 200
_BLOCK_B = 16


def _fused_kernel(idx_ref, tgt_ref, emb_ref, logits_ref, losssum_ref):
    idx_blk = idx_ref[...].astype(jnp.float32)
    tgt_blk = tgt_ref[...].astype(jnp.float32)
    emb = emb_ref[...]
    rb, T = idx_blk.shape
    n = rb * T
    v_pad = emb.shape[1]

    row = jax.lax.broadcasted_iota(jnp.int32, (n, 1), 0)
    colb = jax.lax.broadcasted_iota(jnp.int32, (n, rb), 1)
    sel = ((row // T) == colb).astype(jnp.float32)
    colt = jax.lax.broadcasted_iota(jnp.int32, (n, T), 1)
    tmask = colt == (row % T)

    rows_idx = jnp.dot(sel, idx_blk, preferred_element_type=jnp.float32)
    idx_flat = jnp.sum(jnp.where(tmask, rows_idx, 0.0), axis=1, keepdims=True)
    rows_tgt = jnp.dot(sel, tgt_blk, preferred_element_type=jnp.float32)
    tgt_flat = jnp.sum(jnp.where(tmask, rows_tgt, 0.0), axis=1, keepdims=True)

    colv = jax.lax.broadcasted_iota(jnp.int32, (n, v_pad), 1)
    onehot = (colv == idx_flat.astype(jnp.int32)).astype(jnp.float32)
    logits = jnp.dot(onehot, emb, preferred_element_type=jnp.float32)
    logits_ref[...] = logits

    m = jnp.max(emb, axis=1, keepdims=True)
    lse_vec = m + jnp.log(jnp.sum(jnp.exp(emb - m), axis=1, keepdims=True))
    row_lse = jnp.dot(onehot, lse_vec, preferred_element_type=jnp.float32)

    tgt_logit = jnp.sum(jnp.where(colv == tgt_flat.astype(jnp.int32),
                                  logits, 0.0), axis=1, keepdims=True)
    losssum_ref[...] = jnp.sum(row_lse - tgt_logit, keepdims=True)[None]


@jax.jit
def kernel(idx, targets, emb_padded):
    B, T = idx.shape
    V_pad = emb_padded.shape[1]
    N = B * T
    tile_n = _BLOCK_B * T
    num_tiles = B // _BLOCK_B

    cost = pl.CostEstimate(
        flops=2 * N * V_pad * V_pad,
        transcendentals=num_tiles * V_pad * V_pad,
        bytes_accessed=2 * N * 4 + V_pad * V_pad * 4 + N * _V * 4)
    logits, loss_sums = pl.pallas_call(
        _fused_kernel,
        out_shape=(
            jax.ShapeDtypeStruct((N, V_pad), jnp.float32),
            jax.ShapeDtypeStruct((num_tiles, 1, 1), jnp.float32),
        ),
        grid=(num_tiles,),
        in_specs=[
            pl.BlockSpec((_BLOCK_B, T), lambda i: (i, 0)),
            pl.BlockSpec((_BLOCK_B, T), lambda i: (i, 0)),
            pl.BlockSpec((V_pad, V_pad), lambda i: (0, 0)),
        ],
        out_specs=(
            pl.BlockSpec((tile_n, V_pad), lambda i: (i, 0)),
            pl.BlockSpec((1, 1, 1), lambda i: (i, 0, 0)),
        ),
        compiler_params=pltpu.CompilerParams(
            dimension_semantics=("parallel",),
            vmem_limit_bytes=64 * 1024 * 1024,
        ),
        cost_estimate=cost,
    )(idx, targets, emb_padded)

    loss = jnp.sum(loss_sums) / jnp.float32(N)
    return logits[:, :_V], loss


# speedup vs baseline: 1.7240x; 1.0596x over previous
import jax
import jax.numpy as jnp
from jax.experimental import pallas as pl
from jax.experimental.pallas import tpu as pltpu

_V = 200
_BLOCK_B = 48


def _fused_kernel(idx_ref, tgt_ref, emb_ref, logits_ref, losssum_ref):
    idx_blk = idx_ref[...].astype(jnp.float32)
    tgt_blk = tgt_ref[...].astype(jnp.float32)
    emb = emb_ref[...]
    rb, T = idx_blk.shape
    n = rb * T
    v_pad = emb.shape[1]

    row = jax.lax.broadcasted_iota(jnp.int32, (n, 1), 0)
    colb = jax.lax.broadcasted_iota(jnp.int32, (n, rb), 1)
    sel = ((row // T) == colb).astype(jnp.float32)
    colt = jax.lax.broadcasted_iota(jnp.int32, (n, T), 1)
    tmask = colt == (row % T)

    rows_idx = jnp.dot(sel, idx_blk, preferred_element_type=jnp.float32)
    idx_flat = jnp.sum(jnp.where(tmask, rows_idx, 0.0), axis=1, keepdims=True)
    rows_tgt = jnp.dot(sel, tgt_blk, preferred_element_type=jnp.float32)
    tgt_flat = jnp.sum(jnp.where(tmask, rows_tgt, 0.0), axis=1, keepdims=True)

    colv = jax.lax.broadcasted_iota(jnp.int32, (n, v_pad), 1)
    onehot = (colv == idx_flat.astype(jnp.int32)).astype(jnp.float32)
    logits = jnp.dot(onehot, emb, preferred_element_type=jnp.float32)
    logits_ref[...] = logits

    m = jnp.max(emb, axis=1, keepdims=True)
    lse_vec = m + jnp.log(jnp.sum(jnp.exp(emb - m), axis=1, keepdims=True))
    row_lse = jnp.dot(onehot, lse_vec, preferred_element_type=jnp.float32)

    tgt_logit = jnp.sum(jnp.where(colv == tgt_flat.astype(jnp.int32),
                                  logits, 0.0), axis=1, keepdims=True)
    losssum_ref[...] = jnp.sum(row_lse - tgt_logit, keepdims=True)[None]


@jax.jit
def kernel(idx, targets, emb_padded):
    B, T = idx.shape
    V_pad = emb_padded.shape[1]
    N = B * T
    tile_n = _BLOCK_B * T
    num_tiles = B // _BLOCK_B

    cost = pl.CostEstimate(
        flops=2 * N * V_pad * V_pad,
        transcendentals=num_tiles * V_pad * V_pad,
        bytes_accessed=2 * N * 4 + V_pad * V_pad * 4 + N * _V * 4)
    logits, loss_sums = pl.pallas_call(
        _fused_kernel,
        out_shape=(
            jax.ShapeDtypeStruct((N, V_pad), jnp.float32),
            jax.ShapeDtypeStruct((num_tiles, 1, 1), jnp.float32),
        ),
        grid=(num_tiles,),
        in_specs=[
            pl.BlockSpec((_BLOCK_B, T), lambda i: (i, 0)),
            pl.BlockSpec((_BLOCK_B, T), lambda i: (i, 0)),
            pl.BlockSpec((V_pad, V_pad), lambda i: (0, 0)),
        ],
        out_specs=(
            pl.BlockSpec((tile_n, V_pad), lambda i: (i, 0)),
            pl.BlockSpec((1, 1, 1), lambda i: (i, 0, 0)),
        ),
        compiler_params=pltpu.CompilerParams(
            dimension_semantics=("parallel",),
            vmem_limit_bytes=64 * 1024 * 1024,
        ),
        cost_estimate=cost,
    )(idx, targets, emb_padded)

    loss = jnp.sum(loss_sums) / jnp.float32(N)
    return logits[:, :_V], loss
